# baseline (speedup 1.0000x reference)
import jax
import jax.numpy as jnp
from jax.experimental import pallas as pl
from jax.experimental.pallas import tpu as pltpu

_SQRT_HALF = 0.7071067811865476


def _ffn_kernel(x_ref, fc1_w_ref, fc1_b_ref, ln_g_ref, ln_b_ref,
                fc2_w_ref, fc2_b_ref, o_ref):
    x = x_ref[...].astype(jnp.bfloat16)
    h = jax.lax.dot_general(x, fc1_w_ref[...], (((1,), (1,)), ((), ())),
                            preferred_element_type=jnp.float32)
    h = h + fc1_b_ref[...]
    h = 0.5 * h * (1.0 + jax.lax.erf(h * _SQRT_HALF))
    mu = jnp.mean(h, axis=-1, keepdims=True)
    var = jnp.mean((h - mu) * (h - mu), axis=-1, keepdims=True)
    h = (h - mu) * jax.lax.rsqrt(var + 1e-5) * ln_g_ref[...] + ln_b_ref[...]
    out = jax.lax.dot_general(h.astype(jnp.bfloat16), fc2_w_ref[...],
                              (((1,), (1,)), ((), ())),
                              preferred_element_type=jnp.float32)
    o_ref[...] = out + fc2_b_ref[...]


def kernel(x, gate_w, gate_b, fc1_w, fc1_b, ln_g, ln_b, fc2_w, fc2_b):
    seq_len, batch, d = x.shape
    ffn = fc1_w.shape[0]
    t = seq_len * batch
    x_flat = x.reshape(t, d)

    bt = 256
    grid = (t // bt,)

    out_flat = pl.pallas_call(
        _ffn_kernel,
        grid=grid,
        in_specs=[
            pl.BlockSpec((bt, d), lambda i: (i, 0)),
            pl.BlockSpec((ffn, d), lambda i: (0, 0)),
            pl.BlockSpec((1, ffn), lambda i: (0, 0)),
            pl.BlockSpec((1, ffn), lambda i: (0, 0)),
            pl.BlockSpec((1, ffn), lambda i: (0, 0)),
            pl.BlockSpec((d, ffn), lambda i: (0, 0)),
            pl.BlockSpec((1, d), lambda i: (0, 0)),
        ],
        out_specs=pl.BlockSpec((bt, d), lambda i: (i, 0)),
        out_shape=jax.ShapeDtypeStruct((t, d), jnp.float32),
        compiler_params=pltpu.CompilerParams(
            dimension_semantics=("arbitrary",),
        ),
    )(x_flat, fc1_w.astype(jnp.bfloat16), fc1_b.reshape(1, ffn),
      ln_g.reshape(1, ffn), ln_b.reshape(1, ffn),
      fc2_w.astype(jnp.bfloat16), fc2_b.reshape(1, d))

    output = out_flat.reshape(seq_len, batch, d)
    return (output, jnp.float32(0.0), jnp.float32(0.0))


# speedup vs baseline: 1.1455x; 1.1455x over previous
import jax
import jax.numpy as jnp
from jax.experimental import pallas as pl
from jax.experimental.pallas import tpu as pltpu

_SQRT_HALF = 0.7071067811865476


def _ffn_kernel(x_ref, fc1_w_ref, fc1_b_ref, ln_g_ref, ln_b_ref,
                fc2_w_ref, fc2_b_ref, o_ref):
    x = x_ref[...]
    h = jax.lax.dot_general(x, fc1_w_ref[...], (((1,), (1,)), ((), ())),
                            preferred_element_type=jnp.float32)
    h = h + fc1_b_ref[...]
    h = 0.5 * h * (1.0 + jax.lax.erf(h * _SQRT_HALF))
    mu = jnp.mean(h, axis=-1, keepdims=True)
    var = jnp.mean((h - mu) * (h - mu), axis=-1, keepdims=True)
    h = (h - mu) * jax.lax.rsqrt(var + 1e-5) * ln_g_ref[...] + ln_b_ref[...]
    out = jax.lax.dot_general(h, fc2_w_ref[...], (((1,), (1,)), ((), ())),
                              preferred_element_type=jnp.float32)
    o_ref[...] = out + fc2_b_ref[...]


def kernel(x, gate_w, gate_b, fc1_w, fc1_b, ln_g, ln_b, fc2_w, fc2_b):
    seq_len, batch, d = x.shape
    ffn = fc1_w.shape[0]
    t = seq_len * batch
    x_flat = x.reshape(t, d)

    bt = 512
    grid = (t // bt,)

    out_flat = pl.pallas_call(
        _ffn_kernel,
        grid=grid,
        in_specs=[
            pl.BlockSpec((bt, d), lambda i: (i, 0)),
            pl.BlockSpec((ffn, d), lambda i: (0, 0)),
            pl.BlockSpec((1, ffn), lambda i: (0, 0)),
            pl.BlockSpec((1, ffn), lambda i: (0, 0)),
            pl.BlockSpec((1, ffn), lambda i: (0, 0)),
            pl.BlockSpec((d, ffn), lambda i: (0, 0)),
            pl.BlockSpec((1, d), lambda i: (0, 0)),
        ],
        out_specs=pl.BlockSpec((bt, d), lambda i: (i, 0)),
        out_shape=jax.ShapeDtypeStruct((t, d), jnp.float32),
        compiler_params=pltpu.CompilerParams(
            dimension_semantics=("arbitrary",),
        ),
    )(x_flat, fc1_w, fc1_b.reshape(1, ffn), ln_g.reshape(1, ffn),
      ln_b.reshape(1, ffn), fc2_w, fc2_b.reshape(1, d))

    output = out_flat.reshape(seq_len, batch, d)
    return (output, jnp.float32(0.0), jnp.float32(0.0))
